# baseline (speedup 1.0000x reference)
import functools

import jax
import jax.numpy as jnp
from jax import lax
from jax.experimental import pallas as pl
from jax.experimental.pallas import tpu as pltpu, tpu_sc as plsc

EMBED = 64
B = 4096
L = 200

NC = 2
NS = 16
NW = NC * NS
CHUNK = 100
NCHUNK = (B * L) // (CHUNK * NW)
LANES = 16


def _body(x_hbm, tok_hbm, pos_hbm, out_hbm, idx_v, rows_v, pos_v, sem):
    wid = lax.axis_index("s") * NC + lax.axis_index("c")
    base = wid * NCHUNK
    pltpu.sync_copy(x_hbm.at[pl.ds(base, NCHUNK)], idx_v)
    pltpu.sync_copy(pos_hbm.at[pl.ds(0, 2 * CHUNK)], pos_v)

    def chunk_body(c, carry):
        pltpu.async_copy(tok_hbm.at[idx_v.at[c]], rows_v, sem).wait()
        poff = (c % 2) * CHUNK

        def row_body(r, rcarry):
            for j in range(EMBED // LANES):
                sl = pl.ds(j * LANES, LANES)
                rows_v[r, sl] = rows_v[r, sl] + pos_v[poff + r, sl]
            return rcarry

        lax.fori_loop(0, CHUNK, row_body, 0)
        pltpu.sync_copy(rows_v, out_hbm.at[base + c])
        return carry

    lax.fori_loop(0, NCHUNK, chunk_body, 0)


@jax.jit
def kernel(x, token_table, pos_table):
    xf = x.reshape(NW * NCHUNK, CHUNK)
    mesh = plsc.VectorSubcoreMesh(core_axis_name="c", subcore_axis_name="s")
    run = pl.kernel(
        _body,
        mesh=mesh,
        compiler_params=pltpu.CompilerParams(use_tc_tiling_on_sc=False),
        out_type=jax.ShapeDtypeStruct((NW * NCHUNK, CHUNK, EMBED), jnp.float32),
        scratch_types=[
            pltpu.VMEM((NCHUNK, CHUNK), jnp.int32),
            pltpu.VMEM((CHUNK, EMBED), jnp.float32),
            pltpu.VMEM((2 * CHUNK, EMBED), jnp.float32),
            pltpu.SemaphoreType.DMA,
        ],
    )
    out = run(xf, token_table, pos_table)
    return out.reshape(B, L, EMBED)


# speedup vs baseline: 1.4603x; 1.4603x over previous
import functools

import jax
import jax.numpy as jnp
from jax import lax
from jax.experimental import pallas as pl
from jax.experimental.pallas import tpu as pltpu, tpu_sc as plsc

EMBED = 64
B = 4096
L = 200

NC = 2
NS = 16
NW = NC * NS
CHUNK = 100
NCHUNK = (B * L) // (CHUNK * NW)
LANES = 16


def _add_pos_rows(rows_v, pos_v, poff):
    def row_body(k, rcarry):
        r = k * 2
        for u in range(2):
            for j in range(EMBED // LANES):
                sl = pl.ds(j * LANES, LANES)
                rows_v[r + u, sl] = rows_v[r + u, sl] + pos_v[poff + r + u, sl]
        return rcarry

    lax.fori_loop(0, CHUNK // 2, row_body, 0)


def _body(x_hbm, tok_hbm, pos_hbm, out_hbm, idx_v, rows0, rows1, pos_v,
          sem0, sem1):
    wid = lax.axis_index("s") * NC + lax.axis_index("c")
    base = wid * NCHUNK
    pltpu.sync_copy(x_hbm.at[pl.ds(base, NCHUNK)], idx_v)
    pltpu.sync_copy(pos_hbm.at[pl.ds(0, 2 * CHUNK)], pos_v)

    g0 = pltpu.async_copy(tok_hbm.at[idx_v.at[0]], rows0, sem0)

    def pair_body(i, carry):
        c = i * 2
        g1 = pltpu.async_copy(tok_hbm.at[idx_v.at[c + 1]], rows1, sem1)
        pltpu.make_async_copy(tok_hbm.at[idx_v.at[c]], rows0, sem0).wait()
        _add_pos_rows(rows0, pos_v, 0)
        pltpu.sync_copy(rows0, out_hbm.at[base + c])

        @pl.when(i < NCHUNK // 2 - 1)
        def _():
            pltpu.async_copy(tok_hbm.at[idx_v.at[c + 2]], rows0, sem0)

        pltpu.make_async_copy(tok_hbm.at[idx_v.at[c + 1]], rows1, sem1).wait()
        _add_pos_rows(rows1, pos_v, CHUNK)
        pltpu.sync_copy(rows1, out_hbm.at[base + c + 1])
        return carry

    lax.fori_loop(0, NCHUNK // 2, pair_body, 0)


@jax.jit
def kernel(x, token_table, pos_table):
    xf = x.reshape(NW * NCHUNK, CHUNK)
    mesh = plsc.VectorSubcoreMesh(core_axis_name="c", subcore_axis_name="s")
    run = pl.kernel(
        _body,
        mesh=mesh,
        compiler_params=pltpu.CompilerParams(use_tc_tiling_on_sc=False),
        out_type=jax.ShapeDtypeStruct((NW * NCHUNK, CHUNK, EMBED), jnp.float32),
        scratch_types=[
            pltpu.VMEM((NCHUNK, CHUNK), jnp.int32),
            pltpu.VMEM((CHUNK, EMBED), jnp.float32),
            pltpu.VMEM((CHUNK, EMBED), jnp.float32),
            pltpu.VMEM((2 * CHUNK, EMBED), jnp.float32),
            pltpu.SemaphoreType.DMA,
            pltpu.SemaphoreType.DMA,
        ],
    )
    out = run(xf, token_table, pos_table)
    return out.reshape(B, L, EMBED)


# speedup vs baseline: 1.4642x; 1.0027x over previous
import functools

import jax
import jax.numpy as jnp
from jax import lax
from jax.experimental import pallas as pl
from jax.experimental.pallas import tpu as pltpu, tpu_sc as plsc
from jax.experimental.layout import Format, Layout

EMBED = 64
B = 4096
L = 200

NC = 2
NS = 16
NW = NC * NS
CHUNK = 100
NCHUNK = (B * L) // (CHUNK * NW)
LANES = 16


def _add_pos_rows(rows_v, pos_v, poff):
    def row_body(k, rcarry):
        r = k * 2
        for u in range(2):
            for j in range(EMBED // LANES):
                sl = pl.ds(j * LANES, LANES)
                rows_v[r + u, sl] = rows_v[r + u, sl] + pos_v[poff + r + u, sl]
        return rcarry

    lax.fori_loop(0, CHUNK // 2, row_body, 0)


def _body(x_hbm, tok_hbm, pos_hbm, out_hbm, idx_v, rows0, rows1, pos_v,
          sem0, sem1):
    wid = lax.axis_index("s") * NC + lax.axis_index("c")
    base = wid * NCHUNK
    pltpu.sync_copy(x_hbm.at[pl.ds(base, NCHUNK)], idx_v)
    pltpu.sync_copy(pos_hbm.at[pl.ds(0, 2 * CHUNK)], pos_v)

    pltpu.async_copy(tok_hbm.at[idx_v.at[0]], rows0, sem0)

    def pair_body(i, carry):
        c = i * 2
        pltpu.async_copy(tok_hbm.at[idx_v.at[c + 1]], rows1, sem1)
        pltpu.make_async_copy(tok_hbm.at[idx_v.at[c]], rows0, sem0).wait()
        _add_pos_rows(rows0, pos_v, 0)
        pltpu.sync_copy(rows0, out_hbm.at[base + c])

        @pl.when(i < NCHUNK // 2 - 1)
        def _():
            pltpu.async_copy(tok_hbm.at[idx_v.at[c + 2]], rows0, sem0)

        pltpu.make_async_copy(tok_hbm.at[idx_v.at[c + 1]], rows1, sem1).wait()
        _add_pos_rows(rows1, pos_v, CHUNK)
        pltpu.sync_copy(rows1, out_hbm.at[base + c + 1])
        return carry

    lax.fori_loop(0, NCHUNK // 2, pair_body, 0)


def _impl(x, token_table, pos_table):
    xf = x.reshape(NW * NCHUNK, CHUNK)
    mesh = plsc.VectorSubcoreMesh(core_axis_name="c", subcore_axis_name="s")
    run = pl.kernel(
        _body,
        mesh=mesh,
        compiler_params=pltpu.CompilerParams(
            use_tc_tiling_on_sc=False, needs_layout_passes=False),
        out_type=jax.ShapeDtypeStruct((NW * NCHUNK, CHUNK, EMBED), jnp.float32),
        scratch_types=[
            pltpu.VMEM((NCHUNK, CHUNK), jnp.int32),
            pltpu.VMEM((CHUNK, EMBED), jnp.float32),
            pltpu.VMEM((CHUNK, EMBED), jnp.float32),
            pltpu.VMEM((2 * CHUNK, EMBED), jnp.float32),
            pltpu.SemaphoreType.DMA,
            pltpu.SemaphoreType.DMA,
        ],
    )
    out = run(xf, token_table, pos_table)
    return out.reshape(B, L, EMBED)


_impl.__name__ = "kernel"
_JIT_CACHE = {}


def kernel(x, token_table, pos_table):
    sh = getattr(x, "sharding", None)
    if sh is None:
        sh = jax.sharding.SingleDeviceSharding(jax.devices()[0])
    fn = _JIT_CACHE.get(sh)
    if fn is None:
        fmt = Format(Layout(major_to_minor=(0, 1, 2)), sh)
        fn = jax.jit(_impl, out_shardings=fmt)
        _JIT_CACHE[sh] = fn
    return fn(x, token_table, pos_table)


# speedup vs baseline: 1.4649x; 1.0005x over previous
import functools

import jax
import jax.numpy as jnp
from jax import lax
from jax.experimental import pallas as pl
from jax.experimental.pallas import tpu as pltpu, tpu_sc as plsc
from jax.experimental.layout import Format, Layout

EMBED = 64
B = 4096
L = 200

NC = 2
NS = 16
NW = NC * NS
CHUNK = 100
NCHUNK = (B * L) // (CHUNK * NW)
LANES = 16


def _add_pos_rows(rows_v, pos_v, poff):
    def row_body(k, rcarry):
        r = k * 2
        for u in range(2):
            for j in range(EMBED // LANES):
                sl = pl.ds(j * LANES, LANES)
                rows_v[r + u, sl] = rows_v[r + u, sl] + pos_v[poff + r + u, sl]
        return rcarry

    lax.fori_loop(0, CHUNK // 2, row_body, 0)


def _body(x_hbm, tok_hbm, pos_hbm, out_hbm, idx_v, rows0, rows1, pos_v,
          sem0, sem1):
    wid = lax.axis_index("s") * NC + lax.axis_index("c")
    base = wid * NCHUNK
    pltpu.sync_copy(x_hbm.at[pl.ds(base, NCHUNK)], idx_v)
    pltpu.sync_copy(pos_hbm.at[pl.ds(0, 2 * CHUNK)], pos_v)

    pltpu.async_copy(tok_hbm.at[idx_v.at[0]], rows0, sem0)

    def pair_body(i, carry):
        c = i * 2
        pltpu.async_copy(tok_hbm.at[idx_v.at[c + 1]], rows1, sem1)
        pltpu.make_async_copy(tok_hbm.at[idx_v.at[c]], rows0, sem0).wait()
        _add_pos_rows(rows0, pos_v, 0)
        pltpu.sync_copy(rows0, out_hbm.at[base + c])

        @pl.when(i < NCHUNK // 2 - 1)
        def _():
            pltpu.async_copy(tok_hbm.at[idx_v.at[c + 2]], rows0, sem0)

        pltpu.make_async_copy(tok_hbm.at[idx_v.at[c + 1]], rows1, sem1).wait()
        _add_pos_rows(rows1, pos_v, CHUNK)
        pltpu.sync_copy(rows1, out_hbm.at[base + c + 1])
        return carry

    lax.fori_loop(0, NCHUNK // 2, pair_body, 0)


def _impl(x, token_table, pos_table):
    xf = x.reshape(NW * NCHUNK, CHUNK)
    mesh = plsc.VectorSubcoreMesh(core_axis_name="c", subcore_axis_name="s")
    run = pl.kernel(
        _body,
        mesh=mesh,
        compiler_params=pltpu.CompilerParams(
            use_tc_tiling_on_sc=False, needs_layout_passes=False),
        out_type=jax.ShapeDtypeStruct((NW * NCHUNK, CHUNK, EMBED), jnp.float32),
        scratch_types=[
            pltpu.VMEM((NCHUNK, CHUNK), jnp.int32),
            pltpu.VMEM((CHUNK, EMBED), jnp.float32),
            pltpu.VMEM((CHUNK, EMBED), jnp.float32),
            pltpu.VMEM((2 * CHUNK, EMBED), jnp.float32),
            pltpu.SemaphoreType.DMA,
            pltpu.SemaphoreType.DMA,
        ],
    )
    out = run(xf, token_table, pos_table)
    return out.reshape(B, L, EMBED)


_impl.__name__ = "kernel"
_JIT_CACHE = {}


def kernel(x, token_table, pos_table):
    sh = getattr(x, "sharding", None)
    if sh is None:
        sh = jax.sharding.SingleDeviceSharding(jax.devices()[0])
    fn = _JIT_CACHE.get(sh)
    if fn is None:
        fmt = Format(Layout(major_to_minor=(0, 1, 2), tiling=((8,),)), sh)
        fn = jax.jit(_impl, out_shardings=fmt)
        _JIT_CACHE[sh] = fn
    return fn(x, token_table, pos_table)
